# baseline (speedup 1.0000x reference)
import functools

import jax
import jax.numpy as jnp
from jax import lax
from jax.experimental import pallas as pl
from jax.experimental.pallas import tpu as pltpu
from jax.experimental.pallas import tpu_sc as plsc

VOCAB = 1000
EMBED = 64
BATCH = 4096
FIELDS = 26
B_TOTAL = BATCH * FIELDS
NUM_WORKERS = 32
B_PER_W = B_TOTAL // NUM_WORKERS
CHUNK = 128
N_CHUNKS = B_PER_W // CHUNK
NBUF = 8
G_AHEAD = 4


def _sc_embedding_gather(table, idx2d):
    mesh = plsc.VectorSubcoreMesh(core_axis_name="c", subcore_axis_name="s")

    @functools.partial(
        pl.kernel,
        mesh=mesh,
        out_type=jax.ShapeDtypeStruct((B_TOTAL, EMBED), jnp.float32),
        compiler_params=pltpu.CompilerParams(use_tc_tiling_on_sc=False),
        scratch_types=[
            pltpu.VMEM((N_CHUNKS, CHUNK), jnp.int32),
            pltpu.VMEM((NBUF, CHUNK, EMBED), jnp.float32),
            pltpu.SemaphoreType.DMA,
            pltpu.SemaphoreType.DMA,
        ],
    )
    def k(table_hbm, idx_hbm, out_hbm, idx_v, rows_v, gsem, osem):
        wid = lax.axis_index("s") * 2 + lax.axis_index("c")
        chunk0 = wid * N_CHUNKS

        pltpu.sync_copy(idx_hbm.at[wid], idx_v)

        for g in range(G_AHEAD):
            pltpu.async_copy(table_hbm.at[idx_v.at[g]], rows_v.at[g], gsem)

        def out_copy(g, b):
            return pltpu.make_async_copy(
                rows_v.at[b],
                out_hbm.at[pl.ds((chunk0 + g) * CHUNK, CHUNK)],
                osem,
            )

        def body(g, _):
            b = lax.rem(g, NBUF)
            ng = g + G_AHEAD
            fire = ng < N_CHUNKS

            @pl.when(jnp.logical_and(g >= G_AHEAD, fire))
            def _():
                out_copy(g, b).wait()

            @pl.when(fire)
            def _():
                pltpu.async_copy(
                    table_hbm.at[idx_v.at[ng]],
                    rows_v.at[lax.rem(ng, NBUF)],
                    gsem,
                )

            pltpu.make_async_copy(
                table_hbm.at[idx_v.at[g]], rows_v.at[b], gsem
            ).wait()
            out_copy(g, b).start()
            return ()

        lax.fori_loop(0, N_CHUNKS, body, (), unroll=False)

        for r in range(NBUF):
            out_copy(N_CHUNKS - NBUF + r,
                     (N_CHUNKS - NBUF + r) % NBUF).wait()

    return k(table, idx2d)


def kernel(inputs, table):
    idx3d = inputs.reshape(NUM_WORKERS, N_CHUNKS, CHUNK)
    out = _sc_embedding_gather(table, idx3d)
    return out.reshape(BATCH, FIELDS, EMBED)


# speedup vs baseline: 1.0042x; 1.0042x over previous
import functools

import jax
import jax.numpy as jnp
from jax import lax
from jax.experimental import pallas as pl
from jax.experimental.pallas import tpu as pltpu
from jax.experimental.pallas import tpu_sc as plsc

VOCAB = 1000
EMBED = 64
BATCH = 4096
FIELDS = 26
B_TOTAL = BATCH * FIELDS
NUM_WORKERS = 32
B_PER_W = B_TOTAL // NUM_WORKERS
CHUNK = 128
N_CHUNKS = B_PER_W // CHUNK
NBUF = 8
G_AHEAD = 4


def _sc_embedding_gather(table, idx2d):
    mesh = plsc.VectorSubcoreMesh(core_axis_name="c", subcore_axis_name="s")

    @functools.partial(
        pl.kernel,
        mesh=mesh,
        out_type=jax.ShapeDtypeStruct((B_TOTAL, EMBED), jnp.float32),
        compiler_params=pltpu.CompilerParams(use_tc_tiling_on_sc=False),
        scratch_types=[
            pltpu.VMEM((N_CHUNKS, CHUNK), jnp.int32),
            pltpu.VMEM((NBUF, CHUNK, EMBED), jnp.float32),
            pltpu.SemaphoreType.DMA,
            pltpu.SemaphoreType.DMA,
        ],
    )
    def k(table_hbm, idx_hbm, out_hbm, idx_v, rows_v, gsem, osem):
        wid = lax.axis_index("s") * 2 + lax.axis_index("c")
        chunk0 = wid * N_CHUNKS

        pltpu.sync_copy(idx_hbm.at[wid], idx_v)

        for g in range(G_AHEAD):
            pltpu.async_copy(table_hbm.at[idx_v.at[g]], rows_v.at[g], gsem)

        def out_copy(g, b):
            return pltpu.make_async_copy(
                rows_v.at[b],
                out_hbm.at[pl.ds((chunk0 + g) * CHUNK, CHUNK)],
                osem,
            )

        def body(g, _):
            b = lax.rem(g, NBUF)
            ng = g + G_AHEAD
            fire = ng < N_CHUNKS

            @pl.when(jnp.logical_and(g >= G_AHEAD, fire))
            def _():
                out_copy(g, b).wait()

            @pl.when(fire)
            def _():
                pltpu.async_copy(
                    table_hbm.at[idx_v.at[ng]],
                    rows_v.at[lax.rem(ng, NBUF)],
                    gsem,
                )

            pltpu.make_async_copy(
                table_hbm.at[idx_v.at[g]], rows_v.at[b], gsem
            ).wait()
            out_copy(g, b).start()
            return ()

        lax.fori_loop(0, N_CHUNKS, body, (), unroll=False)

        for r in range(NBUF):
            out_copy(N_CHUNKS - NBUF + r,
                     (N_CHUNKS - NBUF + r) % NBUF).wait()

    return k(table, idx2d)


def kernel(inputs, table):
    idx3d = inputs.reshape(NUM_WORKERS, N_CHUNKS, CHUNK) ^ 0
    out = _sc_embedding_gather(table, idx3d)
    return out.reshape(BATCH, FIELDS, EMBED)
